# baseline (speedup 1.0000x reference)
import jax
import jax.numpy as jnp
from jax import lax
from jax.experimental import pallas as pl
from jax.experimental.pallas import tpu as pltpu
from jax.experimental.pallas import tpu_sc as plsc

NC = 2
NS = 16
NW = NC * NS

B = 16384
S = 50
V = 1_000_000
D = 32

BPW = B // NW
G = 128
NH = BPW // G
NPAIR = S * NH // 2


def _body(idx3, table2, out3, idxq_v, r3_v, rows0, rows1, rows2, rows3,
          tbuf0, tbuf1, g0, g1, g2, g3, o0, o1):
    w = lax.axis_index("s") * NC + lax.axis_index("c")
    iota = lax.iota(jnp.int32, 16)
    b0 = w * BPW

    pltpu.sync_copy(idx3.at[:, pl.ds(4 * w, 4), :], idxq_v)

    def split(s, carry):
        for j in range(NH):
            for u in range(8):
                x = idxq_v[s, j, pl.ds(16 * u, 16)]
                r3_v[s, j, pl.ds(16 * u, 16)] = (x & 3) * D
                idxq_v[s, j, pl.ds(16 * u, 16)] = x >> 2
        return carry

    lax.fori_loop(0, S, split, 0)

    def fire_g(s, h, rows, sem):
        pltpu.async_copy(table2.at[idxq_v.at[s, h]], rows, sem)

    def wait_g(rows, sem):
        pltpu.make_async_copy(table2.at[pl.ds(0, G)], rows, sem).wait()

    def fire_o(s, h, tbuf, sem):
        pltpu.async_copy(tbuf, out3.at[s, :, pl.ds(b0 + G * h, G)], sem)

    def wait_o(tbuf, sem):
        pltpu.make_async_copy(tbuf, out3.at[0, :, pl.ds(0, G)], sem).wait()

    def extract(s, h, rows, tbuf):
        def vbody(v, carry):
            rv = 16 * v + iota
            base = r3_v[s, h, pl.ds(16 * v, 16)]
            for d in range(D):
                x = plsc.load_gather(rows, [rv, base + d])
                tbuf[d, pl.ds(16 * v, 16)] = x
            return carry

        lax.fori_loop(0, G // 16, vbody, 0)

    rows = [rows0, rows1, rows2, rows3]
    gsem = [g0, g1, g2, g3]
    tb = [tbuf0, tbuf1]
    osem = [o0, o1]

    for q in range(4):
        fire_g(0, q, rows[q], gsem[q])

    def step(i, carry):
        for q in range(4):
            wait_g(rows[q], gsem[q])
            if q < 2:
                @pl.when(i > 0)
                def _():
                    wait_o(tb[q], osem[q])
            else:
                wait_o(tb[q - 2], osem[q - 2])
            extract(i, q, rows[q], tb[q % 2])
            fire_o(i, q, tb[q % 2], osem[q % 2])

            @pl.when(i < S - 1)
            def _():
                fire_g(i + 1, q, rows[q], gsem[q])
        return carry

    lax.fori_loop(0, S, step, 0)
    wait_o(tbuf0, o0)
    wait_o(tbuf1, o1)


def kernel(inputs, index, weights):
    table2 = weights.reshape(V // 4, 128)
    idx3 = inputs.T.reshape(S, B // 128, 128)

    p = pl.kernel(
        _body,
        out_type=jax.ShapeDtypeStruct((S, D, B), jnp.float32),
        mesh=plsc.VectorSubcoreMesh(core_axis_name="c", subcore_axis_name="s"),
        compiler_params=pltpu.CompilerParams(use_tc_tiling_on_sc=False,
                                             needs_layout_passes=False),
        scratch_types=[
            pltpu.VMEM((S, NH, 128), jnp.int32),
            pltpu.VMEM((S, NH, 128), jnp.int32),
            pltpu.VMEM((G, 128), jnp.float32),
            pltpu.VMEM((G, 128), jnp.float32),
            pltpu.VMEM((G, 128), jnp.float32),
            pltpu.VMEM((G, 128), jnp.float32),
            pltpu.VMEM((D, G), jnp.float32),
            pltpu.VMEM((D, G), jnp.float32),
            pltpu.SemaphoreType.DMA,
            pltpu.SemaphoreType.DMA,
            pltpu.SemaphoreType.DMA,
            pltpu.SemaphoreType.DMA,
            pltpu.SemaphoreType.DMA,
            pltpu.SemaphoreType.DMA,
        ],
    )
    out3 = p(idx3, table2)
    return out3.transpose(2, 0, 1)
